# Initial kernel scaffold; baseline (speedup 1.0000x reference)
import functools

import jax
import jax.numpy as jnp
from jax import lax
from jax.experimental import pallas as pl
from jax.experimental.pallas import tpu as pltpu
from jax.experimental.pallas import tpu_sc as plsc

_NUM_CORES = 2
_NUM_SUBCORES = 16
_NUM_WORKERS = _NUM_CORES * _NUM_SUBCORES

_CHUNK = 128


def _make_gather(num_rows: int, embed: int):
  assert num_rows % (_NUM_WORKERS * _CHUNK) == 0
  rows_per_worker = num_rows // _NUM_WORKERS
  n_chunks = rows_per_worker // _CHUNK

  mesh = plsc.VectorSubcoreMesh(
      core_axis_name="c", subcore_axis_name="s")

  @functools.partial(
      pl.kernel,
      out_type=jax.ShapeDtypeStruct((num_rows, embed), jnp.float32),
      mesh=mesh,
      scratch_types=[
          pltpu.VMEM((n_chunks, _CHUNK), jnp.int32),
          pltpu.VMEM((_CHUNK, embed), jnp.float32),
          pltpu.SemaphoreType.DMA,
      ],
  )
  def gather_kernel(idx_hbm, table_hbm, out_hbm, idx_v, rows_v, sem):
    wid = lax.axis_index("s") * _NUM_CORES + lax.axis_index("c")
    base = wid * rows_per_worker
    pltpu.sync_copy(idx_hbm.at[wid], idx_v)

    def step(j, carry):
      pltpu.async_copy(table_hbm.at[idx_v.at[j]], rows_v, sem).wait()
      pltpu.sync_copy(rows_v, out_hbm.at[pl.ds(base + j * _CHUNK, _CHUNK)])
      return carry

    lax.fori_loop(0, n_chunks, step, 0)

  return gather_kernel


def kernel(x, table):
  batch, fields = x.shape
  vocab, embed = table.shape
  num_rows = batch * fields
  idx = x.reshape(_NUM_WORKERS, num_rows // (_NUM_WORKERS * _CHUNK), _CHUNK)
  idx = idx.astype(jnp.int32)
  out = _make_gather(num_rows, embed)(idx, table)
  return out.reshape(batch, fields, embed)


# speedup vs baseline: 1.4360x; 1.4360x over previous
import functools

import jax
import jax.numpy as jnp
from jax import lax
from jax.experimental import pallas as pl
from jax.experimental.pallas import tpu as pltpu
from jax.experimental.pallas import tpu_sc as plsc

_NUM_CORES = 2
_NUM_SUBCORES = 16
_NUM_WORKERS = _NUM_CORES * _NUM_SUBCORES

_CHUNK = 128


def _make_gather(num_rows: int, embed: int):
  assert num_rows % (_NUM_WORKERS * _CHUNK) == 0
  rows_per_worker = num_rows // _NUM_WORKERS
  n_chunks = rows_per_worker // _CHUNK

  mesh = plsc.VectorSubcoreMesh(
      core_axis_name="c", subcore_axis_name="s")

  @functools.partial(
      pl.kernel,
      out_type=jax.ShapeDtypeStruct((num_rows, embed), jnp.float32),
      mesh=mesh,
      scratch_types=[
          pltpu.VMEM((n_chunks, _CHUNK), jnp.int32),
          pltpu.VMEM((_CHUNK, embed), jnp.float32),
          pltpu.SemaphoreType.DMA,
      ],
      compiler_params=pltpu.CompilerParams(use_tc_tiling_on_sc=False),
  )
  def gather_kernel(idx_hbm, table_hbm, out_hbm, idx_v, rows_v, sem):
    wid = lax.axis_index("s") * _NUM_CORES + lax.axis_index("c")
    base = wid * rows_per_worker
    pltpu.sync_copy(idx_hbm.at[wid], idx_v)

    def step(j, carry):
      pltpu.async_copy(table_hbm.at[idx_v.at[j]], rows_v, sem).wait()
      pltpu.sync_copy(rows_v, out_hbm.at[pl.ds(base + j * _CHUNK, _CHUNK)])
      return carry

    lax.fori_loop(0, n_chunks, step, 0)

  return gather_kernel


def kernel(x, table):
  batch, fields = x.shape
  vocab, embed = table.shape
  num_rows = batch * fields
  idx = x.reshape(_NUM_WORKERS, num_rows // (_NUM_WORKERS * _CHUNK), _CHUNK)
  idx = idx.astype(jnp.int32)
  out = _make_gather(num_rows, embed)(idx, table)
  return out.reshape(batch, fields, embed)


# speedup vs baseline: 1.5601x; 1.0864x over previous
import functools

import jax
import jax.numpy as jnp
from jax import lax
from jax.experimental import pallas as pl
from jax.experimental.pallas import tpu as pltpu
from jax.experimental.pallas import tpu_sc as plsc

_NUM_CORES = 2
_NUM_SUBCORES = 16
_NUM_WORKERS = _NUM_CORES * _NUM_SUBCORES

_CHUNK = 1024


def _make_gather(num_rows: int, embed: int):
  assert num_rows % (_NUM_WORKERS * _CHUNK) == 0
  rows_per_worker = num_rows // _NUM_WORKERS
  n_chunks = rows_per_worker // _CHUNK

  mesh = plsc.VectorSubcoreMesh(
      core_axis_name="c", subcore_axis_name="s")

  @functools.partial(
      pl.kernel,
      out_type=jax.ShapeDtypeStruct((num_rows, embed), jnp.float32),
      mesh=mesh,
      scratch_types=[
          pltpu.VMEM((n_chunks, _CHUNK), jnp.int32),
          pltpu.VMEM((_CHUNK, embed), jnp.float32),
          pltpu.SemaphoreType.DMA,
      ],
      compiler_params=pltpu.CompilerParams(use_tc_tiling_on_sc=False),
  )
  def gather_kernel(idx_hbm, table_hbm, out_hbm, idx_v, rows_v, sem):
    wid = lax.axis_index("s") * _NUM_CORES + lax.axis_index("c")
    base = wid * rows_per_worker
    pltpu.sync_copy(idx_hbm.at[wid], idx_v)

    def step(j, carry):
      pltpu.async_copy(table_hbm.at[idx_v.at[j]], rows_v, sem).wait()
      pltpu.sync_copy(rows_v, out_hbm.at[pl.ds(base + j * _CHUNK, _CHUNK)])
      return carry

    lax.fori_loop(0, n_chunks, step, 0)

  return gather_kernel


def kernel(x, table):
  batch, fields = x.shape
  vocab, embed = table.shape
  num_rows = batch * fields
  idx = x.reshape(_NUM_WORKERS, num_rows // (_NUM_WORKERS * _CHUNK), _CHUNK)
  idx = idx.astype(jnp.int32)
  out = _make_gather(num_rows, embed)(idx, table)
  return out.reshape(batch, fields, embed)


# speedup vs baseline: 1.5761x; 1.0103x over previous
import functools

import jax
import jax.numpy as jnp
from jax import lax
from jax.experimental import pallas as pl
from jax.experimental.pallas import tpu as pltpu
from jax.experimental.pallas import tpu_sc as plsc

_NUM_CORES = 2
_NUM_SUBCORES = 16
_NUM_WORKERS = _NUM_CORES * _NUM_SUBCORES

_CHUNK = 512
_NBUF = 4


def _make_gather(num_rows: int, embed: int):
  assert num_rows % (_NUM_WORKERS * _CHUNK) == 0
  rows_per_worker = num_rows // _NUM_WORKERS
  n_chunks = rows_per_worker // _CHUNK
  ahead = _NBUF - 1
  assert n_chunks >= ahead

  mesh = plsc.VectorSubcoreMesh(
      core_axis_name="c", subcore_axis_name="s")

  @functools.partial(
      pl.kernel,
      out_type=jax.ShapeDtypeStruct((num_rows, embed), jnp.float32),
      mesh=mesh,
      scratch_types=[
          pltpu.VMEM((n_chunks, _CHUNK), jnp.int32),
          pltpu.VMEM((_NBUF, _CHUNK, embed), jnp.float32),
          pltpu.SemaphoreType.DMA((_NBUF,)),
          pltpu.SemaphoreType.DMA((_NBUF,)),
      ],
      compiler_params=pltpu.CompilerParams(use_tc_tiling_on_sc=False),
  )
  def gather_kernel(idx_hbm, table_hbm, out_hbm, idx_v, rows_v, gsem, osem):
    wid = lax.axis_index("s") * _NUM_CORES + lax.axis_index("c")
    base = wid * rows_per_worker
    pltpu.sync_copy(idx_hbm.at[wid], idx_v)

    for j in range(ahead):
      pltpu.async_copy(table_hbm.at[idx_v.at[j]], rows_v.at[j], gsem.at[j])

    def step(j, carry):
      b = lax.rem(j, _NBUF)
      bp = lax.rem(j + _NBUF - 1, _NBUF)
      pltpu.make_async_copy(
          table_hbm.at[idx_v.at[j]], rows_v.at[b], gsem.at[b]).wait()

      @pl.when(j >= 1)
      def _():
        pltpu.make_async_copy(
            rows_v.at[bp],
            out_hbm.at[pl.ds(base + (j - 1) * _CHUNK, _CHUNK)],
            osem.at[bp]).wait()

      @pl.when(j + ahead < n_chunks)
      def _():
        pltpu.async_copy(
            table_hbm.at[idx_v.at[j + ahead]], rows_v.at[bp], gsem.at[bp])

      pltpu.async_copy(
          rows_v.at[b], out_hbm.at[pl.ds(base + j * _CHUNK, _CHUNK)],
          osem.at[b])
      return carry

    lax.fori_loop(0, n_chunks, step, 0)

    bl = (n_chunks - 1) % _NBUF
    pltpu.make_async_copy(
        rows_v.at[bl],
        out_hbm.at[pl.ds(base + (n_chunks - 1) * _CHUNK, _CHUNK)],
        osem.at[bl]).wait()

  return gather_kernel


def kernel(x, table):
  batch, fields = x.shape
  vocab, embed = table.shape
  num_rows = batch * fields
  idx = x.reshape(_NUM_WORKERS, num_rows // (_NUM_WORKERS * _CHUNK), _CHUNK)
  idx = idx.astype(jnp.int32)
  out = _make_gather(num_rows, embed)(idx, table)
  return out.reshape(batch, fields, embed)
